# baseline (speedup 1.0000x reference)
import functools

import numpy as np
import jax
import jax.numpy as jnp
from jax.experimental import pallas as pl

_NPOINT = [2048, 512, 128, 32]
_RADII = [[0.005, 0.01, 0.015], [0.02, 0.03, 0.04], [0.05, 0.1], [0.1, 0.15]]
_NSAMPLES = [[8, 16, 32], [8, 16, 32], [16, 32], [16, 32]]
_BN_SCALE = np.float32(1.0 / np.sqrt(1.0 + 1e-05))
_F32 = jnp.float32


def _sqdist_block(q, pt):
    qx, qy, qz = q[:, 0:1], q[:, 1:2], q[:, 2:3]
    qq = (qx * qx + qy * qy) + qz * qz
    px, py, pz = pt[0:1, :], pt[1:2, :], pt[2:3, :]
    pp = (px * px + py * py) + pz * pz
    qp = jnp.dot(q, pt, preferred_element_type=_F32)
    return (qq + pp) - 2.0 * qp


def _fps_body(npoint, x_ref, o_ref):
    B = x_ref.shape[0]
    N = x_ref.shape[2]
    px = x_ref[:, 0, :]
    py = x_ref[:, 1, :]
    pz = x_ref[:, 2, :]
    iota = jax.lax.broadcasted_iota(jnp.int32, (B, N), 1)
    iop = jax.lax.broadcasted_iota(jnp.int32, (B, npoint), 1)

    def step(i, st):
        dist, far, cx, cy, cz = st
        selmask = iota == far
        sx = jnp.sum(jnp.where(selmask, px, 0.0), axis=1, keepdims=True)
        sy = jnp.sum(jnp.where(selmask, py, 0.0), axis=1, keepdims=True)
        sz = jnp.sum(jnp.where(selmask, pz, 0.0), axis=1, keepdims=True)
        cx = jnp.where(iop == i, sx, cx)
        cy = jnp.where(iop == i, sy, cy)
        cz = jnp.where(iop == i, sz, cz)
        dx = px - sx
        dy = py - sy
        dz = pz - sz
        d = (dx * dx + dy * dy) + dz * dz
        dist = jnp.minimum(dist, d)
        m = jnp.max(dist, axis=1, keepdims=True)
        far = jnp.min(jnp.where(dist == m, iota, N), axis=1, keepdims=True)
        return dist, far, cx, cy, cz

    dist0 = jnp.full((B, N), 1e10, _F32)
    far0 = jnp.zeros((B, 1), jnp.int32)
    c0 = jnp.zeros((B, npoint), _F32)
    _, _, cx, cy, cz = jax.lax.fori_loop(0, npoint, step,
                                         (dist0, far0, c0, c0, c0))
    o_ref[:, 0, :] = cx
    o_ref[:, 1, :] = cy
    o_ref[:, 2, :] = cz


def _fps(xyz_t, npoint):
    B = xyz_t.shape[0]
    return pl.pallas_call(
        functools.partial(_fps_body, npoint),
        out_shape=jax.ShapeDtypeStruct((B, 3, npoint), _F32),
    )(xyz_t)


def _sa_scale_body(r2, ns, nl, q_ref, pt_ref, pf_ref, *refs):
    o_ref = refs[-1]
    wrefs = refs[:-1]
    q = q_ref[0]
    pt = pt_ref[0]
    pf = pf_ref[0]
    Sb = q.shape[0]
    N = pt.shape[1]
    Cin = pf.shape[1]
    Cout = o_ref.shape[2]

    d2 = _sqdist_block(q, pt)
    iota = jax.lax.broadcasted_iota(jnp.int32, (Sb, N), 1)
    keys0 = jnp.where(d2 <= r2, iota, N)
    if Cin > 3:
        qpad = jnp.concatenate([q, jnp.zeros((Sb, Cin - 3), _F32)], axis=1)
    else:
        qpad = q

    m0 = jnp.min(keys0, axis=1, keepdims=True)
    first = jnp.where(m0 >= N, 0, m0)

    def cond(st):
        t, _, m, _ = st
        alive = jnp.logical_or(t == 0, jnp.min(m) < N)
        return jnp.logical_and(t < ns, alive)

    def body(st):
        t, keys, m, acc = st
        sel = jnp.where(m >= N, first, m)
        oh = (iota == sel).astype(_F32)
        g = jnp.dot(oh, pf, precision='highest', preferred_element_type=_F32)
        h = g - qpad
        for i in range(nl):
            w = wrefs[2 * i][...]
            b = wrefs[2 * i + 1][...]
            h = jnp.dot(h, w, preferred_element_type=_F32) + b
            h = jnp.maximum(h * _BN_SCALE, 0.0)
        acc = jnp.where(t == 0, h, jnp.maximum(acc, h))
        keys = jnp.where(keys == m, N, keys)
        m = jnp.min(keys, axis=1, keepdims=True)
        return t + 1, keys, m, acc

    init = (jnp.int32(0), keys0, m0, jnp.zeros((Sb, Cout), _F32))
    _, _, _, acc = jax.lax.while_loop(cond, body, init)
    o_ref[0] = acc


def _sa_scale(new_r, xyz_t, pf, radius, ns, layers):
    B, S, _ = new_r.shape
    N = xyz_t.shape[2]
    Cin = pf.shape[2]
    r2 = np.float32(radius * radius)
    wts = []
    for (W, b) in layers:
        wts.append(jnp.transpose(W))
        wts.append(b.reshape(1, -1))
    Cout = layers[-1][0].shape[0]
    Sb = min(S, 64)
    grid = (B, S // Sb)
    in_specs = [
        pl.BlockSpec((1, Sb, 3), lambda b, s: (b, s, 0)),
        pl.BlockSpec((1, 3, N), lambda b, s: (b, 0, 0)),
        pl.BlockSpec((1, N, Cin), lambda b, s: (b, 0, 0)),
    ]
    for w in wts:
        in_specs.append(pl.BlockSpec(w.shape, lambda b, s: tuple(0 for _ in w.shape)))
    body = functools.partial(_sa_scale_body, r2, ns, len(layers))
    return pl.pallas_call(
        body,
        grid=grid,
        in_specs=in_specs,
        out_specs=pl.BlockSpec((1, Sb, Cout), lambda b, s: (b, s, 0)),
        out_shape=jax.ShapeDtypeStruct((B, S, Cout), _F32),
    )(new_r, xyz_t, pf, *wts)


def _sa_level(xyz_r, xyz_t, feats_r, npoint, radii, nsamples, level_params):
    new_t = _fps(xyz_t, npoint)
    new_r = jnp.transpose(new_t, (0, 2, 1))
    if feats_r is None:
        pf = xyz_r
    else:
        pf = jnp.concatenate([xyz_r, feats_r], axis=2)
    outs = []
    for radius, ns, layers in zip(radii, nsamples, level_params):
        outs.append(_sa_scale(new_r, xyz_t, pf, radius, ns, layers))
    return new_r, new_t, jnp.concatenate(outs, axis=2)


def _fp_body(nl, has_f1, q_ref, pt_ref, f2_ref, *refs):
    if has_f1:
        f1_ref = refs[0]
        refs = refs[1:]
    o_ref = refs[-1]
    wrefs = refs[:-1]
    q = q_ref[0]
    pt = pt_ref[0]
    f2 = f2_ref[0]
    Nb = q.shape[0]
    N2 = pt.shape[1]

    cur = _sqdist_block(q, pt)
    iota = jax.lax.broadcasted_iota(jnp.int32, (Nb, N2), 1)
    gs, ms = [], []
    for _ in range(3):
        m = jnp.min(cur, axis=1, keepdims=True)
        idx = jnp.min(jnp.where(cur == m, iota, N2), axis=1, keepdims=True)
        oh = (iota == idx).astype(_F32)
        gs.append(jnp.dot(oh, f2, precision='highest',
                          preferred_element_type=_F32))
        ms.append(m)
        cur = jnp.where(iota == idx, jnp.float32(3.0e38), cur)

    r0 = 1.0 / (ms[0] + 1e-08)
    r1 = 1.0 / (ms[1] + 1e-08)
    r2 = 1.0 / (ms[2] + 1e-08)
    s = (r0 + r1) + r2
    interp = (gs[0] * (r0 / s) + gs[1] * (r1 / s)) + gs[2] * (r2 / s)
    if has_f1:
        h = jnp.concatenate([interp, f1_ref[0]], axis=1)
    else:
        h = interp
    for i in range(nl):
        w = wrefs[2 * i][...]
        b = wrefs[2 * i + 1][...]
        h = jnp.dot(h, w, preferred_element_type=_F32) + b
        h = jnp.maximum(h * _BN_SCALE, 0.0)
    o_ref[0] = h


def _fp(xyz1_r, feats1_r, xyz2_t, feats2_r, layers):
    B, N1, _ = xyz1_r.shape
    N2 = xyz2_t.shape[2]
    C2 = feats2_r.shape[2]
    wts = []
    for (W, b) in layers:
        wts.append(jnp.transpose(W))
        wts.append(b.reshape(1, -1))
    Cout = layers[-1][0].shape[0]
    Nb = min(N1, 512)
    grid = (B, N1 // Nb)
    in_specs = [
        pl.BlockSpec((1, Nb, 3), lambda b, n: (b, n, 0)),
        pl.BlockSpec((1, 3, N2), lambda b, n: (b, 0, 0)),
        pl.BlockSpec((1, N2, C2), lambda b, n: (b, 0, 0)),
    ]
    args = [xyz1_r, xyz2_t, feats2_r]
    if feats1_r is not None:
        C1 = feats1_r.shape[2]
        in_specs.append(pl.BlockSpec((1, Nb, C1), lambda b, n: (b, n, 0)))
        args.append(feats1_r)
    for w in wts:
        in_specs.append(pl.BlockSpec(w.shape, lambda b, n: tuple(0 for _ in w.shape)))
    args.extend(wts)
    body = functools.partial(_fp_body, len(layers), feats1_r is not None)
    return pl.pallas_call(
        body,
        grid=grid,
        in_specs=in_specs,
        out_specs=pl.BlockSpec((1, Nb, Cout), lambda b, n: (b, n, 0)),
        out_shape=jax.ShapeDtypeStruct((B, N1, Cout), _F32),
    )(*args)


def kernel(pointcloud, params):
    xyz_r = pointcloud[..., 0:3]
    xyz_t = jnp.transpose(xyz_r, (0, 2, 1))
    l_xyz_r = [xyz_r]
    l_xyz_t = [xyz_t]
    l_feats = [None]
    for lvl in range(4):
        new_r, new_t, f_r = _sa_level(
            l_xyz_r[lvl], l_xyz_t[lvl], l_feats[lvl], _NPOINT[lvl],
            _RADII[lvl], _NSAMPLES[lvl], params['sa'][lvl])
        l_xyz_r.append(new_r)
        l_xyz_t.append(new_t)
        l_feats.append(f_r)
    for i in range(-1, -5, -1):
        l_feats[i - 1] = _fp(l_xyz_r[i - 1], l_feats[i - 1], l_xyz_t[i],
                             l_feats[i], params['fp'][i])
    return jnp.transpose(l_feats[0], (0, 2, 1))


# speedup vs baseline: 1.4881x; 1.4881x over previous
import functools

import numpy as np
import jax
import jax.numpy as jnp
from jax.experimental import pallas as pl

_NPOINT = [2048, 512, 128, 32]
_RADII = [[0.005, 0.01, 0.015], [0.02, 0.03, 0.04], [0.05, 0.1], [0.1, 0.15]]
_NSAMPLES = [[8, 16, 32], [8, 16, 32], [16, 32], [16, 32]]
_BN_SCALE = np.float32(1.0 / np.sqrt(1.0 + 1e-05))
_F32 = jnp.float32


def _split_bf16(x):
    hi = x.astype(jnp.bfloat16)
    r1 = x - hi.astype(_F32)
    mid = r1.astype(jnp.bfloat16)
    lo = (r1 - mid.astype(_F32)).astype(jnp.bfloat16)
    return hi, mid, lo


def _sqdist_block(q, pt):
    qx, qy, qz = q[:, 0:1], q[:, 1:2], q[:, 2:3]
    qq = (qx * qx + qy * qy) + qz * qz
    px, py, pz = pt[0:1, :], pt[1:2, :], pt[2:3, :]
    pp = (px * px + py * py) + pz * pz
    qp = jnp.dot(q, pt, preferred_element_type=_F32)
    return (qq + pp) - 2.0 * qp


def _fps_body(npoint, x_ref, o_ref):
    B = x_ref.shape[0]
    N = x_ref.shape[2]
    px = x_ref[:, 0, :]
    py = x_ref[:, 1, :]
    pz = x_ref[:, 2, :]
    iota = jax.lax.broadcasted_iota(jnp.int32, (B, N), 1)
    iop = jax.lax.broadcasted_iota(jnp.int32, (B, npoint), 1)

    def step(i, st):
        dist, far, cx, cy, cz = st
        selmask = iota == far
        sx = jnp.sum(jnp.where(selmask, px, 0.0), axis=1, keepdims=True)
        sy = jnp.sum(jnp.where(selmask, py, 0.0), axis=1, keepdims=True)
        sz = jnp.sum(jnp.where(selmask, pz, 0.0), axis=1, keepdims=True)
        cx = jnp.where(iop == i, sx, cx)
        cy = jnp.where(iop == i, sy, cy)
        cz = jnp.where(iop == i, sz, cz)
        dx = px - sx
        dy = py - sy
        dz = pz - sz
        d = (dx * dx + dy * dy) + dz * dz
        dist = jnp.minimum(dist, d)
        m = jnp.max(dist, axis=1, keepdims=True)
        far = jnp.min(jnp.where(dist == m, iota, N), axis=1, keepdims=True)
        return dist, far, cx, cy, cz

    dist0 = jnp.full((B, N), 1e10, _F32)
    far0 = jnp.zeros((B, 1), jnp.int32)
    c0 = jnp.zeros((B, npoint), _F32)
    _, _, cx, cy, cz = jax.lax.fori_loop(0, npoint, step,
                                         (dist0, far0, c0, c0, c0))
    o_ref[:, 0, :] = cx
    o_ref[:, 1, :] = cy
    o_ref[:, 2, :] = cz


def _fps(xyz_t, npoint):
    B = xyz_t.shape[0]
    return pl.pallas_call(
        functools.partial(_fps_body, npoint),
        out_shape=jax.ShapeDtypeStruct((B, 3, npoint), _F32),
    )(xyz_t)


def _sa_scale_body(r2, ns, nl, q_ref, pt_ref, pfh_ref, pfm_ref, pfl_ref, *refs):
    o_ref = refs[-1]
    wrefs = refs[:-1]
    q = q_ref[0]
    pt = pt_ref[0]
    pfh = pfh_ref[0]
    pfm = pfm_ref[0]
    pfl = pfl_ref[0]
    Sb = q.shape[0]
    N = pt.shape[1]
    Cin = pfh.shape[1]
    Cout = o_ref.shape[2]

    d2 = _sqdist_block(q, pt)
    iota = jax.lax.broadcasted_iota(jnp.int32, (Sb, N), 1)
    keys0 = jnp.where(d2 <= r2, iota, N)
    if Cin > 3:
        qpad = jnp.concatenate([q, jnp.zeros((Sb, Cin - 3), _F32)], axis=1)
    else:
        qpad = q

    m0 = jnp.min(keys0, axis=1, keepdims=True)
    first = jnp.where(m0 >= N, 0, m0)

    def cond(st):
        t, _, m, _ = st
        alive = jnp.logical_or(t == 0, jnp.min(m) < N)
        return jnp.logical_and(t < ns, alive)

    def body(st):
        t, keys, m, acc = st
        sel = jnp.where(m >= N, first, m)
        oh = (iota == sel).astype(jnp.bfloat16)
        g = (jnp.dot(oh, pfh, preferred_element_type=_F32) +
             jnp.dot(oh, pfm, preferred_element_type=_F32) +
             jnp.dot(oh, pfl, preferred_element_type=_F32))
        h = g - qpad
        for i in range(nl):
            w = wrefs[2 * i][...]
            b = wrefs[2 * i + 1][...]
            h = jnp.dot(h, w, preferred_element_type=_F32) + b
            h = jnp.maximum(h * _BN_SCALE, 0.0)
        acc = jnp.where(t == 0, h, jnp.maximum(acc, h))
        keys = jnp.where(keys == m, N, keys)
        m = jnp.min(keys, axis=1, keepdims=True)
        return t + 1, keys, m, acc

    init = (jnp.int32(0), keys0, m0, jnp.zeros((Sb, Cout), _F32))
    _, _, _, acc = jax.lax.while_loop(cond, body, init)
    o_ref[0] = acc


def _sa_scale(new_r, xyz_t, pf, radius, ns, layers):
    B, S, _ = new_r.shape
    N = xyz_t.shape[2]
    Cin = pf.shape[2]
    pfh, pfm, pfl = _split_bf16(pf)
    r2 = np.float32(radius * radius)
    wts = []
    for (W, b) in layers:
        wts.append(jnp.transpose(W))
        wts.append(b.reshape(1, -1))
    Cout = layers[-1][0].shape[0]
    Sb = min(S, 128)
    grid = (B, S // Sb)
    in_specs = [
        pl.BlockSpec((1, Sb, 3), lambda b, s: (b, s, 0)),
        pl.BlockSpec((1, 3, N), lambda b, s: (b, 0, 0)),
        pl.BlockSpec((1, N, Cin), lambda b, s: (b, 0, 0)),
        pl.BlockSpec((1, N, Cin), lambda b, s: (b, 0, 0)),
        pl.BlockSpec((1, N, Cin), lambda b, s: (b, 0, 0)),
    ]
    for w in wts:
        in_specs.append(pl.BlockSpec(w.shape, lambda b, s: tuple(0 for _ in w.shape)))
    body = functools.partial(_sa_scale_body, r2, ns, len(layers))
    return pl.pallas_call(
        body,
        grid=grid,
        in_specs=in_specs,
        out_specs=pl.BlockSpec((1, Sb, Cout), lambda b, s: (b, s, 0)),
        out_shape=jax.ShapeDtypeStruct((B, S, Cout), _F32),
    )(new_r, xyz_t, pfh, pfm, pfl, *wts)


def _sa_level(xyz_r, xyz_t, feats_r, npoint, radii, nsamples, level_params):
    new_t = _fps(xyz_t, npoint)
    new_r = jnp.transpose(new_t, (0, 2, 1))
    if feats_r is None:
        pf = xyz_r
    else:
        pf = jnp.concatenate([xyz_r, feats_r], axis=2)
    outs = []
    for radius, ns, layers in zip(radii, nsamples, level_params):
        outs.append(_sa_scale(new_r, xyz_t, pf, radius, ns, layers))
    return new_r, new_t, jnp.concatenate(outs, axis=2)


def _fp_body(nl, has_f1, q_ref, pt_ref, f2h_ref, f2m_ref, f2l_ref, *refs):
    if has_f1:
        f1_ref = refs[0]
        refs = refs[1:]
    o_ref = refs[-1]
    wrefs = refs[:-1]
    q = q_ref[0]
    pt = pt_ref[0]
    f2h = f2h_ref[0]
    f2m = f2m_ref[0]
    f2l = f2l_ref[0]
    Nb = q.shape[0]
    N2 = pt.shape[1]

    cur = _sqdist_block(q, pt)
    iota = jax.lax.broadcasted_iota(jnp.int32, (Nb, N2), 1)
    gs, ms = [], []
    for _ in range(3):
        m = jnp.min(cur, axis=1, keepdims=True)
        idx = jnp.min(jnp.where(cur == m, iota, N2), axis=1, keepdims=True)
        oh = (iota == idx).astype(jnp.bfloat16)
        gs.append(jnp.dot(oh, f2h, preferred_element_type=_F32) +
                  jnp.dot(oh, f2m, preferred_element_type=_F32) +
                  jnp.dot(oh, f2l, preferred_element_type=_F32))
        ms.append(m)
        cur = jnp.where(iota == idx, jnp.float32(3.0e38), cur)

    r0 = 1.0 / (ms[0] + 1e-08)
    r1 = 1.0 / (ms[1] + 1e-08)
    r2 = 1.0 / (ms[2] + 1e-08)
    s = (r0 + r1) + r2
    interp = (gs[0] * (r0 / s) + gs[1] * (r1 / s)) + gs[2] * (r2 / s)
    if has_f1:
        h = jnp.concatenate([interp, f1_ref[0]], axis=1)
    else:
        h = interp
    for i in range(nl):
        w = wrefs[2 * i][...]
        b = wrefs[2 * i + 1][...]
        h = jnp.dot(h, w, preferred_element_type=_F32) + b
        h = jnp.maximum(h * _BN_SCALE, 0.0)
    o_ref[0] = h


def _fp(xyz1_r, feats1_r, xyz2_t, feats2_r, layers):
    B, N1, _ = xyz1_r.shape
    N2 = xyz2_t.shape[2]
    C2 = feats2_r.shape[2]
    f2h, f2m, f2l = _split_bf16(feats2_r)
    wts = []
    for (W, b) in layers:
        wts.append(jnp.transpose(W))
        wts.append(b.reshape(1, -1))
    Cout = layers[-1][0].shape[0]
    Nb = min(N1, 512)
    grid = (B, N1 // Nb)
    in_specs = [
        pl.BlockSpec((1, Nb, 3), lambda b, n: (b, n, 0)),
        pl.BlockSpec((1, 3, N2), lambda b, n: (b, 0, 0)),
        pl.BlockSpec((1, N2, C2), lambda b, n: (b, 0, 0)),
        pl.BlockSpec((1, N2, C2), lambda b, n: (b, 0, 0)),
        pl.BlockSpec((1, N2, C2), lambda b, n: (b, 0, 0)),
    ]
    args = [xyz1_r, xyz2_t, f2h, f2m, f2l]
    if feats1_r is not None:
        C1 = feats1_r.shape[2]
        in_specs.append(pl.BlockSpec((1, Nb, C1), lambda b, n: (b, n, 0)))
        args.append(feats1_r)
    for w in wts:
        in_specs.append(pl.BlockSpec(w.shape, lambda b, n: tuple(0 for _ in w.shape)))
    args.extend(wts)
    body = functools.partial(_fp_body, len(layers), feats1_r is not None)
    return pl.pallas_call(
        body,
        grid=grid,
        in_specs=in_specs,
        out_specs=pl.BlockSpec((1, Nb, Cout), lambda b, n: (b, n, 0)),
        out_shape=jax.ShapeDtypeStruct((B, N1, Cout), _F32),
    )(*args)


def kernel(pointcloud, params):
    xyz_r = pointcloud[..., 0:3]
    xyz_t = jnp.transpose(xyz_r, (0, 2, 1))
    l_xyz_r = [xyz_r]
    l_xyz_t = [xyz_t]
    l_feats = [None]
    for lvl in range(4):
        new_r, new_t, f_r = _sa_level(
            l_xyz_r[lvl], l_xyz_t[lvl], l_feats[lvl], _NPOINT[lvl],
            _RADII[lvl], _NSAMPLES[lvl], params['sa'][lvl])
        l_xyz_r.append(new_r)
        l_xyz_t.append(new_t)
        l_feats.append(f_r)
    for i in range(-1, -5, -1):
        l_feats[i - 1] = _fp(l_xyz_r[i - 1], l_feats[i - 1], l_xyz_t[i],
                             l_feats[i], params['fp'][i])
    return jnp.transpose(l_feats[0], (0, 2, 1))
